# baseline (speedup 1.0000x reference)
import functools

import jax
import jax.numpy as jnp
from jax import lax
from jax.experimental import pallas as pl
from jax.experimental.pallas import tpu as pltpu
from jax.experimental.pallas import tpu_sc as plsc

N = 10000
D = 128
G = 64
NC = 2
NS = 16
NW = NC * NS
CB = 125
NP = 4
CHP = 20
CH = NP * CHP
E_CAP = CB * CH * NW
RPT = 624
REM = N - NS * RPT

_mesh = plsc.VectorSubcoreMesh(core_axis_name="c", subcore_axis_name="s")


@functools.partial(
    pl.kernel,
    mesh=_mesh,
    out_type=jax.ShapeDtypeStruct((2 * N, D), jnp.float32),
    scratch_types=[
        pltpu.VMEM_SHARED((N, D), jnp.float32),
        pltpu.VMEM((CHP, CB), jnp.int32),
        pltpu.VMEM((CHP, CB), jnp.int32),
        pltpu.VMEM((CHP, CB), jnp.int32),
        pltpu.VMEM((CHP, CB), jnp.int32),
        pltpu.VMEM((CB, D), jnp.float32),
        pltpu.VMEM((CB, D), jnp.float32),
        pltpu.SemaphoreType.DMA,
        pltpu.SemaphoreType.DMA,
        pltpu.SemaphoreType.DMA,
        pltpu.SemaphoreType.DMA,
        pltpu.SemaphoreType.DMA,
    ],
)
def _sc_aggregate(h_hbm, src_hbm, dst_hbm, out_hbm, accum, src_a, dst_a,
                  src_b, dst_b, rows0, rows1, gsem0, gsem1, isem_a, isem_b,
                  hsem):
    c = lax.axis_index("c")
    s = lax.axis_index("s")
    wid = s * NC + c
    r0 = s * RPT
    bufs = ((src_a, dst_a, isem_a), (src_b, dst_b, isem_b))

    def stage(p, sv, dv, isem):
        pltpu.async_copy(src_hbm.at[wid * NP + p], sv, isem)
        pltpu.async_copy(dst_hbm.at[wid * NP + p], dv, isem)

    def stage_wait(p, sv, dv, isem):
        pltpu.make_async_copy(src_hbm.at[wid * NP + p], sv, isem).wait()
        pltpu.make_async_copy(dst_hbm.at[wid * NP + p], dv, isem).wait()

    stage(0, *bufs[0])
    pltpu.async_copy(h_hbm.at[pl.ds(r0, RPT)], accum.at[pl.ds(r0, RPT)],
                     hsem)

    @pl.when(s == NS - 1)
    def _init_rem():
        pltpu.async_copy(h_hbm.at[pl.ds(NS * RPT, REM)],
                         accum.at[pl.ds(NS * RPT, REM)], hsem)

    pltpu.make_async_copy(h_hbm.at[pl.ds(r0, RPT)],
                          accum.at[pl.ds(r0, RPT)], hsem).wait()

    @pl.when(s == NS - 1)
    def _init_rem_wait():
        pltpu.make_async_copy(h_hbm.at[pl.ds(NS * RPT, REM)],
                              accum.at[pl.ds(NS * RPT, REM)], hsem).wait()

    plsc.subcore_barrier()

    for p in range(NP):
        sv, dv, isem = bufs[p % 2]
        stage_wait(p, sv, dv, isem)
        if p + 1 < NP:
            stage(p + 1, *bufs[(p + 1) % 2])

        def gather(j, buf, sem):
            pltpu.async_copy(h_hbm.at[sv.at[j]], buf, sem)

        def gwait(j, buf, sem):
            pltpu.make_async_copy(h_hbm.at[sv.at[j]], buf, sem).wait()

        gather(0, rows0, gsem0)
        gather(1, rows1, gsem1)

        def body(k, carry):
            c0 = 2 * k
            gwait(c0, rows0, gsem0)
            pltpu.sync_copy(rows0, accum.at[dv.at[c0]], add=True)

            @pl.when(c0 + 2 < CHP)
            def _():
                gather(c0 + 2, rows0, gsem0)

            gwait(c0 + 1, rows1, gsem1)
            pltpu.sync_copy(rows1, accum.at[dv.at[c0 + 1]], add=True)

            @pl.when(c0 + 3 < CHP)
            def _():
                gather(c0 + 3, rows1, gsem1)

            return carry

        lax.fori_loop(0, CHP // 2, body, 0)
    plsc.subcore_barrier()
    pltpu.sync_copy(accum.at[pl.ds(r0, RPT)],
                    out_hbm.at[pl.ds(c * N + r0, RPT)])

    @pl.when(s == NS - 1)
    def _out_rem():
        pltpu.sync_copy(accum.at[pl.ds(NS * RPT, REM)],
                        out_hbm.at[pl.ds(c * N + NS * RPT, REM)])


def _mlp_body(parts_ref, hp_ref, w1_ref, b1_ref, g1_ref, be1_ref,
              w2_ref, b2_ref, g2_ref, be2_ref, ids_ref, hn_ref, pool_ref):
    agg = parts_ref[:N, :] + parts_ref[N:, :] - hp_ref[...]

    def block(v, w_ref, b_ref, g_ref, be_ref):
        y = jnp.dot(v, w_ref[...], preferred_element_type=jnp.float32,
                    precision=lax.Precision.DEFAULT) + b_ref[...]
        mu = jnp.mean(y, axis=0, keepdims=True)
        var = jnp.mean(jnp.square(y - mu), axis=0, keepdims=True)
        yn = g_ref[...] * (y - mu) * lax.rsqrt(var + 1e-5) + be_ref[...]
        return jnp.maximum(yn, 0.0)

    h1 = block(agg, w1_ref, b1_ref, g1_ref, be1_ref)
    h2 = block(h1, w2_ref, b2_ref, g2_ref, be2_ref)
    hn_ref[...] = h2
    ids = ids_ref[...]
    gidx = lax.broadcasted_iota(jnp.int32, (G, 1), 0)
    oh = (ids == gidx).astype(jnp.float32)
    pool_ref[...] = jnp.dot(oh, h2, preferred_element_type=jnp.float32,
                            precision=lax.Precision.DEFAULT)


_mlp = pl.pallas_call(
    _mlp_body,
    out_shape=[
        jax.ShapeDtypeStruct((N, D), jnp.float32),
        jax.ShapeDtypeStruct((G, D), jnp.float32),
    ],
)


def kernel(x, edge_index, seq_batch_node_id, W0, b0, g0, be0, Ws, bs, gs,
           bes):
    src = edge_index[0].astype(jnp.int32)
    dst = edge_index[1].astype(jnp.int32)
    src_p = src.reshape(NW * NP, CHP, CB)
    dst_p = dst.reshape(NW * NP, CHP, CB)
    ids = seq_batch_node_id.astype(jnp.int32).reshape(1, N)
    r1 = lambda v: v.reshape(1, D)

    h = x
    pools = []
    for w1, b1, g1, be1 in ((W0, b0, g0, be0), (Ws, bs, gs, bes),
                            (Ws, bs, gs, bes)):
        parts = _sc_aggregate(h, src_p, dst_p)
        h, pool = _mlp(parts, h, w1, r1(b1), r1(g1), r1(be1),
                       Ws, r1(bs), r1(gs), r1(bes), ids)
        pools.append(pool)
    return jnp.concatenate(pools, axis=1)


# speedup vs baseline: 1.0273x; 1.0273x over previous
import functools

import jax
import jax.numpy as jnp
from jax import lax
from jax.experimental import pallas as pl
from jax.experimental.pallas import tpu as pltpu
from jax.experimental.pallas import tpu_sc as plsc

N = 10000
D = 128
G = 64
NC = 2
NS = 16
NW = NC * NS
CB = 125
NP = 4
CHP = 20
CH = NP * CHP
E_CAP = CB * CH * NW
RPT = 624
REM = N - NS * RPT

_mesh = plsc.VectorSubcoreMesh(core_axis_name="c", subcore_axis_name="s")


@functools.partial(
    pl.kernel,
    mesh=_mesh,
    out_type=jax.ShapeDtypeStruct((2 * N, D), jnp.float32),
    scratch_types=[
        pltpu.VMEM_SHARED((N, D), jnp.float32),
        pltpu.VMEM((CHP, CB), jnp.int32),
        pltpu.VMEM((CHP, CB), jnp.int32),
        pltpu.VMEM((CHP, CB), jnp.int32),
        pltpu.VMEM((CHP, CB), jnp.int32),
        pltpu.VMEM((CB, D), jnp.float32),
        pltpu.VMEM((CB, D), jnp.float32),
        pltpu.SemaphoreType.DMA,
        pltpu.SemaphoreType.DMA,
        pltpu.SemaphoreType.DMA,
        pltpu.SemaphoreType.DMA,
        pltpu.SemaphoreType.DMA,
    ],
)
def _sc_aggregate(h_hbm, e_hbm, out_hbm, accum, src_a, dst_a,
                  src_b, dst_b, rows0, rows1, gsem0, gsem1, isem_a, isem_b,
                  hsem):
    c = lax.axis_index("c")
    s = lax.axis_index("s")
    wid = s * NC + c
    r0 = s * RPT
    bufs = ((src_a, dst_a, isem_a), (src_b, dst_b, isem_b))

    def stage(p, sv, dv, isem):
        pltpu.async_copy(e_hbm.at[0, wid * NP + p], sv, isem)
        pltpu.async_copy(e_hbm.at[1, wid * NP + p], dv, isem)

    def stage_wait(p, sv, dv, isem):
        pltpu.make_async_copy(e_hbm.at[0, wid * NP + p], sv, isem).wait()
        pltpu.make_async_copy(e_hbm.at[1, wid * NP + p], dv, isem).wait()

    stage(0, *bufs[0])
    pltpu.async_copy(h_hbm.at[pl.ds(r0, RPT)], accum.at[pl.ds(r0, RPT)],
                     hsem)

    @pl.when(s == NS - 1)
    def _init_rem():
        pltpu.async_copy(h_hbm.at[pl.ds(NS * RPT, REM)],
                         accum.at[pl.ds(NS * RPT, REM)], hsem)

    pltpu.make_async_copy(h_hbm.at[pl.ds(r0, RPT)],
                          accum.at[pl.ds(r0, RPT)], hsem).wait()

    @pl.when(s == NS - 1)
    def _init_rem_wait():
        pltpu.make_async_copy(h_hbm.at[pl.ds(NS * RPT, REM)],
                              accum.at[pl.ds(NS * RPT, REM)], hsem).wait()

    plsc.subcore_barrier()

    for p in range(NP):
        sv, dv, isem = bufs[p % 2]
        stage_wait(p, sv, dv, isem)
        if p + 1 < NP:
            stage(p + 1, *bufs[(p + 1) % 2])

        def gather(j, buf, sem):
            pltpu.async_copy(h_hbm.at[sv.at[j]], buf, sem)

        def gwait(j, buf, sem):
            pltpu.make_async_copy(h_hbm.at[sv.at[j]], buf, sem).wait()

        gather(0, rows0, gsem0)
        gather(1, rows1, gsem1)

        def body(k, carry):
            c0 = 2 * k
            gwait(c0, rows0, gsem0)
            pltpu.sync_copy(rows0, accum.at[dv.at[c0]], add=True)

            @pl.when(c0 + 2 < CHP)
            def _():
                gather(c0 + 2, rows0, gsem0)

            gwait(c0 + 1, rows1, gsem1)
            pltpu.sync_copy(rows1, accum.at[dv.at[c0 + 1]], add=True)

            @pl.when(c0 + 3 < CHP)
            def _():
                gather(c0 + 3, rows1, gsem1)

            return carry

        lax.fori_loop(0, CHP // 2, body, 0)
    plsc.subcore_barrier()
    pltpu.sync_copy(accum.at[pl.ds(r0, RPT)],
                    out_hbm.at[pl.ds(c * N + r0, RPT)])

    @pl.when(s == NS - 1)
    def _out_rem():
        pltpu.sync_copy(accum.at[pl.ds(NS * RPT, REM)],
                        out_hbm.at[pl.ds(c * N + NS * RPT, REM)])


def _mlp_body(parts_ref, hp_ref, w1_ref, b1_ref, g1_ref, be1_ref,
              w2_ref, b2_ref, g2_ref, be2_ref, ids_ref, hn_ref, pool_ref):
    agg = parts_ref[:N, :] + parts_ref[N:, :] - hp_ref[...]

    def block(v, w_ref, b_ref, g_ref, be_ref):
        y = jnp.dot(v, w_ref[...], preferred_element_type=jnp.float32,
                    precision=lax.Precision.DEFAULT) + b_ref[...]
        mu = jnp.mean(y, axis=0, keepdims=True)
        var = jnp.mean(jnp.square(y - mu), axis=0, keepdims=True)
        yn = g_ref[...] * (y - mu) * lax.rsqrt(var + 1e-5) + be_ref[...]
        return jnp.maximum(yn, 0.0)

    h1 = block(agg, w1_ref, b1_ref, g1_ref, be1_ref)
    h2 = block(h1, w2_ref, b2_ref, g2_ref, be2_ref)
    hn_ref[...] = h2
    ids = ids_ref[...]
    gidx = lax.broadcasted_iota(jnp.int32, (G, 1), 0)
    oh = (ids == gidx).astype(jnp.float32)
    pool_ref[...] = jnp.dot(oh, h2, preferred_element_type=jnp.float32,
                            precision=lax.Precision.DEFAULT)


_mlp = pl.pallas_call(
    _mlp_body,
    out_shape=[
        jax.ShapeDtypeStruct((N, D), jnp.float32),
        jax.ShapeDtypeStruct((G, D), jnp.float32),
    ],
)


def kernel(x, edge_index, seq_batch_node_id, W0, b0, g0, be0, Ws, bs, gs,
           bes):
    eidx = edge_index.astype(jnp.int32).reshape(2, NW * NP, CHP, CB)
    ids = seq_batch_node_id.astype(jnp.int32).reshape(1, N)
    r1 = lambda v: v.reshape(1, D)

    h = x
    pools = []
    for w1, b1, g1, be1 in ((W0, b0, g0, be0), (Ws, bs, gs, bes),
                            (Ws, bs, gs, bes)):
        parts = _sc_aggregate(h, eidx)
        h, pool = _mlp(parts, h, w1, r1(b1), r1(g1), r1(be1),
                       Ws, r1(bs), r1(gs), r1(bes), ids)
        pools.append(pool)
    return jnp.concatenate(pools, axis=1)
